# baseline (speedup 1.0000x reference)
import functools

import jax
import jax.numpy as jnp
from jax import lax
from jax.experimental import pallas as pl
from jax.experimental.pallas import tpu as pltpu
from jax.experimental.pallas import tpu_sc as plsc

_INV_TS = 10.0
_INV_TT = 25.0

_SUB = 64
_LANE = 128
_G = 8

_NC = 2
_NS = 16
_NW = _NC * _NS
_L = 16


def _compact_body(BN, mask_hbm, idx_hbm, cnt_hbm, mask_v, pos_v, val_v,
                  tot_v, sem):
    chunk = BN // _NW
    nvec = BN // _L
    wid = lax.axis_index("s") * _NC + lax.axis_index("c")

    pltpu.sync_copy(mask_hbm, mask_v)

    def acc_body(k, a):
        return a + mask_v[pl.ds(k * _L, _L)]

    zeros = jnp.zeros((_L,), jnp.int32)
    my_first_vec = wid * (chunk // _L)
    acc = lax.fori_loop(0, my_first_vec, acc_body, zeros)
    base = jnp.sum(acc)
    acc = lax.fori_loop(my_first_vec, nvec, acc_body, acc)
    total = jnp.sum(acc)

    iota = lax.iota(jnp.int32, _L)
    runm = base
    runu = total + wid * chunk - base
    nhalf = chunk // _L // 2
    for half in range(2):
        for j in range(nhalf):
            vj = my_first_vec + half * nhalf + j
            v = mask_v[pl.ds(vj * _L, _L)]
            cums = jnp.cumsum(v)
            nm = jnp.sum(v)
            act = v > 0
            pos = jnp.where(act, runm + cums - 1, runu + (iota + 1 - cums) - 1)
            gid = vj * _L + iota
            val = jnp.where(act, gid, 0)
            pos_v[pl.ds(j * _L, _L)] = pos
            val_v[pl.ds(j * _L, _L)] = val
            runm = runm + nm
            runu = runu + _L - nm
        pltpu.async_copy(val_v, idx_hbm.at[pos_v], sem).wait()

    @pl.when(wid == 0)
    def _write_total():
        tot_v[...] = jnp.full((_L,), total, jnp.int32)
        pltpu.sync_copy(tot_v, cnt_hbm)


def _compact_sc(mask_flat_i32):
    BN = mask_flat_i32.shape[0]
    chunk = BN // _NW
    mesh = plsc.VectorSubcoreMesh(core_axis_name="c", subcore_axis_name="s")
    f = functools.partial(
        pl.kernel,
        mesh=mesh,
        compiler_params=pltpu.CompilerParams(needs_layout_passes=False),
        out_type=[
            jax.ShapeDtypeStruct((BN,), jnp.int32),
            jax.ShapeDtypeStruct((_L,), jnp.int32),
        ],
        scratch_types=[
            pltpu.VMEM((BN,), jnp.int32),
            pltpu.VMEM((chunk // 2,), jnp.int32),
            pltpu.VMEM((chunk // 2,), jnp.int32),
            pltpu.VMEM((_L,), jnp.int32),
            pltpu.SemaphoreType.DMA,
        ],
    )(functools.partial(_compact_body, BN))
    return f(mask_flat_i32)


def _loss_body(idx_ref, cnt_ref, *refs):
    s_refs = refs[:_G]
    t_refs = refs[_G:2 * _G]
    c_ref = refs[2 * _G]
    out_ref = refs[2 * _G + 1]
    acc_ref = refs[2 * _G + 2]

    i = pl.program_id(0)

    @pl.when(i == 0)
    def _init():
        acc_ref[0] = 0.0

    cnt = cnt_ref[0]
    c = c_ref[...]
    for j in range(_G):
        @pl.when(i * _G + j < cnt)
        def _row(j=j):
            t = t_refs[j][...]
            s = s_refs[j][...]
            z = (t - c) * _INV_TT
            zmax = jnp.max(z)
            e = jnp.exp(z - zmax)
            esum = jnp.sum(e)
            y = s * _INV_TS
            ymax = jnp.max(y)
            ysum = jnp.sum(jnp.exp(y - ymax))
            dot = jnp.sum(e * y)
            acc_ref[0] += -(dot / esum) + ymax + jnp.log(ysum)

    @pl.when(i == pl.num_programs(0) - 1)
    def _fin():
        out_ref[0] = acc_ref[0] / jnp.maximum(cnt.astype(jnp.float32), 1.0)


def _loss_tc(idx, cnt, s3, t3, c3):
    BN = s3.shape[0]
    n_steps = BN // _G

    def row_spec(j):
        return pl.BlockSpec(
            (1, _SUB, _LANE),
            lambda i, idx_ref, cnt_ref, j=j: (idx_ref[i * _G + j], 0, 0))

    grid_spec = pltpu.PrefetchScalarGridSpec(
        num_scalar_prefetch=2,
        grid=(n_steps,),
        in_specs=(
            [row_spec(j) for j in range(_G)]
            + [row_spec(j) for j in range(_G)]
            + [pl.BlockSpec((1, _SUB, _LANE),
                            lambda i, idx_ref, cnt_ref: (0, 0, 0))]
        ),
        out_specs=pl.BlockSpec(memory_space=pltpu.SMEM),
        scratch_shapes=[pltpu.SMEM((1,), jnp.float32)],
    )
    out = pl.pallas_call(
        _loss_body,
        grid_spec=grid_spec,
        out_shape=jax.ShapeDtypeStruct((1,), jnp.float32),
    )(idx, cnt, *([s3] * _G), *([t3] * _G), c3)
    return out[0]


def kernel(student_patch_out, teacher_patch_out, mask, center):
    B, N, D = student_patch_out.shape
    BN = B * N
    s3 = student_patch_out.reshape(BN, _SUB, _LANE)
    t3 = teacher_patch_out.reshape(BN, _SUB, _LANE)
    c3 = center.reshape(1, _SUB, _LANE)
    mask_flat = mask.reshape(BN).astype(jnp.int32)
    idx, cnt16 = _compact_sc(mask_flat)
    return _loss_tc(idx, cnt16[0:1], s3, t3, c3)


# speedup vs baseline: 5.2378x; 5.2378x over previous
import jax
import jax.numpy as jnp
from jax.experimental import pallas as pl
from jax.experimental.pallas import tpu as pltpu

_INV_TS = 10.0
_INV_TT = 25.0

_RN = 16


def _loss_body(mask_ref, s_ref, t_ref, c_ref, out_ref, acc_ref, nacc_ref):
    i = pl.program_id(0)

    @pl.when(i == 0)
    def _init():
        acc_ref[0] = 0.0
        nacc_ref[0] = 0.0

    s = s_ref[...]
    t = t_ref[...]
    c = c_ref[...]
    m = mask_ref[...]

    z = (t - c) * _INV_TT
    zmax = jnp.max(z, axis=2, keepdims=True)
    e = jnp.exp(z - zmax)
    esum = jnp.sum(e, axis=2, keepdims=True)

    y = s * _INV_TS
    ymax = jnp.max(y, axis=2, keepdims=True)
    ysum = jnp.sum(jnp.exp(y - ymax), axis=2, keepdims=True)

    dot = jnp.sum(e * y, axis=2, keepdims=True)
    per_token = -(dot / esum) + ymax + jnp.log(ysum)

    acc_ref[0] += jnp.sum(per_token * m)
    nacc_ref[0] += jnp.sum(m)

    @pl.when(i == pl.num_programs(0) - 1)
    def _fin():
        out_ref[0] = acc_ref[0] / jnp.maximum(nacc_ref[0], 1.0)


def kernel(student_patch_out, teacher_patch_out, mask, center):
    B, N, D = student_patch_out.shape
    n_steps = B * N // _RN
    c3 = center.reshape(1, 1, D)
    m3 = mask.reshape(B, N, 1).astype(jnp.float32)

    def row_map(i):
        return (i // (N // _RN), i % (N // _RN), 0)

    out = pl.pallas_call(
        _loss_body,
        grid=(n_steps,),
        in_specs=[
            pl.BlockSpec((1, _RN, 1), row_map),
            pl.BlockSpec((1, _RN, D), row_map),
            pl.BlockSpec((1, _RN, D), row_map),
            pl.BlockSpec((1, 1, D), lambda i: (0, 0, 0)),
        ],
        out_specs=pl.BlockSpec(memory_space=pltpu.SMEM),
        out_shape=jax.ShapeDtypeStruct((1,), jnp.float32),
        scratch_shapes=[
            pltpu.SMEM((1,), jnp.float32),
            pltpu.SMEM((1,), jnp.float32),
        ],
    )(m3, student_patch_out, teacher_patch_out, c3)
    return out[0]


# speedup vs baseline: 5.3588x; 1.0231x over previous
import jax
import jax.numpy as jnp
from jax.experimental import pallas as pl
from jax.experimental.pallas import tpu as pltpu

_INV_TS = 10.0
_INV_TT = 25.0

_RN = 16


_CH = 256


def _loss_body(mask_ref, s_ref, t_ref, c_ref, out_ref, acc_ref, nacc_ref):
    i = pl.program_id(0)

    @pl.when(i == 0)
    def _init():
        acc_ref[0] = 0.0
        nacc_ref[0] = 0.0

    _, RN, D = s_ref.shape
    nch = D // _CH
    m = mask_ref[...]

    tm = jnp.full((1, RN, _CH), -jnp.inf, jnp.float32)
    sm = jnp.full((1, RN, _CH), -jnp.inf, jnp.float32)
    for k in range(nch):
        sl = pl.ds(k * _CH, _CH)
        tm = jnp.maximum(tm, t_ref[:, :, sl] - c_ref[:, :, sl])
        sm = jnp.maximum(sm, s_ref[:, :, sl])
    zmax = _INV_TT * jnp.max(tm, axis=2, keepdims=True)
    ymax = _INV_TS * jnp.max(sm, axis=2, keepdims=True)

    es = jnp.zeros((1, RN, _CH), jnp.float32)
    ss = jnp.zeros((1, RN, _CH), jnp.float32)
    dt = jnp.zeros((1, RN, _CH), jnp.float32)
    for k in range(nch):
        sl = pl.ds(k * _CH, _CH)
        t = t_ref[:, :, sl]
        s = s_ref[:, :, sl]
        c = c_ref[:, :, sl]
        e = jnp.exp((t - c) * _INV_TT - zmax)
        es = es + e
        dt = dt + e * s
        ss = ss + jnp.exp(s * _INV_TS - ymax)
    esum = jnp.sum(es, axis=2, keepdims=True)
    ssum = jnp.sum(ss, axis=2, keepdims=True)
    dot = _INV_TS * jnp.sum(dt, axis=2, keepdims=True)
    per_token = -(dot / esum) + ymax + jnp.log(ssum)

    acc_ref[0] += jnp.sum(per_token * m)
    nacc_ref[0] += jnp.sum(m)

    @pl.when(i == pl.num_programs(0) - 1)
    def _fin():
        out_ref[0] = acc_ref[0] / jnp.maximum(nacc_ref[0], 1.0)


def kernel(student_patch_out, teacher_patch_out, mask, center):
    B, N, D = student_patch_out.shape
    n_steps = B * N // _RN
    c3 = center.reshape(1, 1, D)
    m3 = mask.reshape(B, N, 1).astype(jnp.float32)

    def row_map(i):
        return (i // (N // _RN), i % (N // _RN), 0)

    out = pl.pallas_call(
        _loss_body,
        grid=(n_steps,),
        in_specs=[
            pl.BlockSpec((1, _RN, 1), row_map),
            pl.BlockSpec((1, _RN, D), row_map),
            pl.BlockSpec((1, _RN, D), row_map),
            pl.BlockSpec((1, 1, D), lambda i: (0, 0, 0)),
        ],
        out_specs=pl.BlockSpec(memory_space=pltpu.SMEM),
        out_shape=jax.ShapeDtypeStruct((1,), jnp.float32),
        scratch_shapes=[
            pltpu.SMEM((1,), jnp.float32),
            pltpu.SMEM((1,), jnp.float32),
        ],
    )(m3, student_patch_out, teacher_patch_out, c3)
    return out[0]
